# baseline (speedup 1.0000x reference)
import functools

import jax
import jax.numpy as jnp
from jax.experimental import pallas as pl

_BLOCK_T = 512


def _grounder_block(x_ref, w1_ref, b1_ref, w2_ref, b2_ref,
                    logits_ref, probs_ref, acts_ref):
    x = x_ref[...]
    h = jnp.dot(x, w1_ref[...], preferred_element_type=jnp.float32)
    h = jnp.maximum(h + b1_ref[...], 0.0)
    logits = jnp.dot(h, w2_ref[...], preferred_element_type=jnp.float32)
    logits = logits + b2_ref[...]
    probs = jax.nn.sigmoid(logits)
    logits_ref[...] = logits
    probs_ref[...] = probs
    acts_ref[...] = (probs > 0.5).astype(jnp.float32)


@functools.partial(jax.jit, static_argnames=())
def kernel(neural_repr, W1, b1, W2, b2):
    tokens, embed = neural_repr.shape
    hidden = W1.shape[1]
    symbols = W2.shape[1]
    block_t = _BLOCK_T

    b1_2d = b1.reshape(1, hidden)
    b2_2d = b2.reshape(1, symbols)

    grid = (tokens // block_t,)
    out_shape = [jax.ShapeDtypeStruct((tokens, symbols), jnp.float32)] * 3
    outs = pl.pallas_call(
        _grounder_block,
        grid=grid,
        in_specs=[
            pl.BlockSpec((block_t, embed), lambda i: (i, 0)),
            pl.BlockSpec((embed, hidden), lambda i: (0, 0)),
            pl.BlockSpec((1, hidden), lambda i: (0, 0)),
            pl.BlockSpec((hidden, symbols), lambda i: (0, 0)),
            pl.BlockSpec((1, symbols), lambda i: (0, 0)),
        ],
        out_specs=[
            pl.BlockSpec((block_t, symbols), lambda i: (i, 0)),
            pl.BlockSpec((block_t, symbols), lambda i: (i, 0)),
            pl.BlockSpec((block_t, symbols), lambda i: (i, 0)),
        ],
        out_shape=out_shape,
    )(neural_repr, W1, b1_2d, W2, b2_2d)
    return (outs[0], outs[1], outs[2])


# speedup vs baseline: 1.0398x; 1.0398x over previous
import functools

import jax
import jax.numpy as jnp
from jax.experimental import pallas as pl

_BLOCK_T = 1024


def _grounder_block(x_ref, w1_ref, b1_ref, w2_ref, b2_ref,
                    logits_ref, probs_ref, acts_ref):
    x = x_ref[...]
    h = jnp.dot(x, w1_ref[...], preferred_element_type=jnp.float32)
    h = jnp.maximum(h + b1_ref[...], 0.0)
    logits = jnp.dot(h, w2_ref[...], preferred_element_type=jnp.float32)
    logits = logits + b2_ref[...]
    probs = jax.nn.sigmoid(logits)
    logits_ref[...] = logits
    probs_ref[...] = probs
    acts_ref[...] = (probs > 0.5).astype(jnp.float32)


@functools.partial(jax.jit, static_argnames=())
def kernel(neural_repr, W1, b1, W2, b2):
    tokens, embed = neural_repr.shape
    hidden = W1.shape[1]
    symbols = W2.shape[1]
    block_t = _BLOCK_T

    b1_2d = b1.reshape(1, hidden)
    b2_2d = b2.reshape(1, symbols)

    grid = (tokens // block_t,)
    out_shape = [jax.ShapeDtypeStruct((tokens, symbols), jnp.float32)] * 3
    outs = pl.pallas_call(
        _grounder_block,
        grid=grid,
        in_specs=[
            pl.BlockSpec((block_t, embed), lambda i: (i, 0)),
            pl.BlockSpec((embed, hidden), lambda i: (0, 0)),
            pl.BlockSpec((1, hidden), lambda i: (0, 0)),
            pl.BlockSpec((hidden, symbols), lambda i: (0, 0)),
            pl.BlockSpec((1, symbols), lambda i: (0, 0)),
        ],
        out_specs=[
            pl.BlockSpec((block_t, symbols), lambda i: (i, 0)),
            pl.BlockSpec((block_t, symbols), lambda i: (i, 0)),
            pl.BlockSpec((block_t, symbols), lambda i: (i, 0)),
        ],
        out_shape=out_shape,
    )(neural_repr, W1, b1_2d, W2, b2_2d)
    return (outs[0], outs[1], outs[2])
